# baseline (speedup 1.0000x reference)
import functools

import jax
import jax.numpy as jnp
from jax import lax
from jax.experimental import pallas as pl
from jax.experimental.pallas import tpu as pltpu
from jax.experimental.pallas import tpu_sc as plsc

VOCAB = 1000000
EMBED = 64
HIDDEN = 128
NCLS = 50
B = 4096
L = 200

NC = 2
NS = 16
NW = NC * NS
ROWS_PER_W = B // NW
TOK_PER_W = ROWS_PER_W * L
CH = 640
NCHUNKS = TOK_PER_W // CH
NB = 2
UNROLL = 10
TRIPS = NCHUNKS // UNROLL
ROWS_PER_TRIP = UNROLL * CH // L
VPR = EMBED // 16


def _phase_segments(p):
    t0, t1 = p * CH, (p + 1) * CH
    segs = []
    s = t0
    while s < t1:
        e = min((s // L + 1) * L, t1)
        segs.append((s - t0, e - s, e % L == 0, (e - 1) // L))
        s = e
    return segs


def _sc_pool_body(x_hbm, table_hbm, out_hbm, idx_v, rows_v, out_v,
                  sg0, sg1, si0, si1):
    wid = lax.axis_index("s") * NC + lax.axis_index("c")
    base = wid * TOK_PER_W
    sg = (sg0, sg1)
    si = (si0, si1)

    def issue_idx(c, slot):
        pltpu.async_copy(x_hbm.at[pl.ds(base + c * CH, CH)],
                         idx_v.at[slot], si[slot])

    def wait_idx(slot):
        pltpu.make_async_copy(x_hbm.at[pl.ds(0, CH)], idx_v.at[slot],
                              si[slot]).wait()

    def issue_gather(slot):
        pltpu.async_copy(table_hbm.at[idx_v.at[slot]], rows_v.at[slot],
                         sg[slot])

    def wait_gather(slot):
        pltpu.make_async_copy(table_hbm.at[idx_v.at[slot]], rows_v.at[slot],
                              sg[slot]).wait()

    issue_idx(0, 0)
    wait_idx(0)
    issue_idx(1, 1)
    issue_gather(0)

    zero = jnp.zeros((16,), jnp.float32)
    segs_by_phase = [_phase_segments(p) for p in range(UNROLL)]

    def trip_body(trip, acc):
        row_base = trip * ROWS_PER_TRIP
        for p in range(UNROLL):
            c = trip * UNROLL + p
            slot = p % NB
            nslot = (p + 1) % NB
            wait_gather(slot)

            @pl.when(c + 1 < NCHUNKS)
            def _():
                wait_idx(nslot)
                issue_gather(nslot)

            @pl.when(c + 2 < NCHUNKS)
            def _():
                issue_idx(c + 2, slot)

            for (off, ln, completes, rowb) in segs_by_phase[p]:
                def tok(t, a, _off=off, _slot=slot):
                    ts = _off + t * 8
                    a = list(a)
                    for k in range(8):
                        g = (k & 1) * VPR
                        for j in range(VPR):
                            a[g + j] = a[g + j] + rows_v[
                                _slot, ts + k, pl.ds(j * 16, 16)]
                    return tuple(a)

                acc = lax.fori_loop(0, ln // 8, tok, acc)
                if completes:
                    for j in range(VPR):
                        out_v[row_base + rowb, pl.ds(j * 16, 16)] = (
                            acc[j] + acc[VPR + j])
                    acc = (zero,) * (2 * VPR)
        return acc

    lax.fori_loop(0, TRIPS, trip_body, (zero,) * (2 * VPR))
    pltpu.sync_copy(out_v, out_hbm.at[wid])


def _sc_pool(x_flat, table):
    mesh = plsc.VectorSubcoreMesh(core_axis_name="c", subcore_axis_name="s")
    f = functools.partial(
        pl.kernel,
        out_type=jax.ShapeDtypeStruct((NW, ROWS_PER_W, EMBED), jnp.float32),
        mesh=mesh,
        scratch_types=[
            pltpu.VMEM((NB, CH), jnp.int32),
            pltpu.VMEM((NB, CH, EMBED), jnp.float32),
            pltpu.VMEM((ROWS_PER_W, EMBED), jnp.float32),
        ] + [pltpu.SemaphoreType.DMA] * (2 * NB),
        compiler_params=pltpu.CompilerParams(use_tc_tiling_on_sc=False),
    )(_sc_pool_body)
    return f(x_flat, table)


def _mlp_body(e_ref, inv_ref, wh_ref, bh_ref, wf_ref, bf_ref, o_ref):
    e = e_ref[...] * inv_ref[...]
    e = jnp.where(e > 0, e, jnp.exp(e) - 1.0)
    h = lax.dot_general(e, wh_ref[...], (((1,), (1,)), ((), ())),
                        preferred_element_type=jnp.float32) + bh_ref[...]
    h = jnp.where(h > 0, h, jnp.exp(h) - 1.0)
    o = lax.dot_general(h, wf_ref[...], (((1,), (1,)), ((), ())),
                        preferred_element_type=jnp.float32) + bf_ref[...]
    m = jnp.max(o, axis=1, keepdims=True)
    o = o - m
    s = jnp.log(jnp.sum(jnp.exp(o), axis=1, keepdims=True))
    o_ref[...] = o - s


def _tc_mlp(pooled, inv_len, W_h, b_h, W_f, b_f):
    return pl.pallas_call(
        _mlp_body,
        out_shape=jax.ShapeDtypeStruct((B, NCLS), jnp.float32),
    )(pooled, inv_len, W_h, b_h, W_f, b_f)


def kernel(x, x_len, table, W_h, b_h, W_f, b_f):
    x_flat = x.reshape(NW, TOK_PER_W).reshape(NW * TOK_PER_W)
    pooled = _sc_pool(x_flat, table).reshape(B, EMBED)
    inv_len = (1.0 / x_len.astype(jnp.float32)).reshape(B, 1)
    return _tc_mlp(pooled, inv_len, W_h, b_h.reshape(1, HIDDEN),
                   W_f, b_f.reshape(1, NCLS))


# speedup vs baseline: 1.0133x; 1.0133x over previous
import functools

import jax
import jax.numpy as jnp
from jax import lax
from jax.experimental import pallas as pl
from jax.experimental.pallas import tpu as pltpu
from jax.experimental.pallas import tpu_sc as plsc

VOCAB = 1000000
EMBED = 64
HIDDEN = 128
NCLS = 50
B = 4096
L = 200

NC = 2
NS = 16
NW = NC * NS
ROWS_PER_W = B // NW
CR = 4
NCHUNKS = ROWS_PER_W // CR
NB = 2
VPR = EMBED // 16


def _sc_pool_body(x_hbm, table_hbm, out_hbm, idx_v, rows_v, out_v,
                  sg0, sg1, si0, si1):
    wid = lax.axis_index("s") * NC + lax.axis_index("c")
    row0 = wid * ROWS_PER_W
    sg = (sg0, sg1)
    si = (si0, si1)

    def issue_idx(c, slot):
        pltpu.async_copy(x_hbm.at[pl.ds(row0 + c * CR, CR)],
                         idx_v.at[slot], si[slot])

    def wait_idx(slot):
        pltpu.make_async_copy(x_hbm.at[pl.ds(0, CR)], idx_v.at[slot],
                              si[slot]).wait()

    def issue_gather(slot):
        for r in range(CR):
            pltpu.async_copy(table_hbm.at[idx_v.at[slot, r]],
                             rows_v.at[slot, r], sg[slot])

    def wait_gather(slot):
        for r in range(CR):
            pltpu.make_async_copy(table_hbm.at[idx_v.at[slot, r]],
                                  rows_v.at[slot, r], sg[slot]).wait()

    issue_idx(0, 0)
    wait_idx(0)
    issue_idx(1, 1)
    issue_gather(0)

    zero = jnp.zeros((16,), jnp.float32)

    def trip_body(trip, _):
        for p in range(NB):
            c = trip * NB + p
            slot = p
            nslot = (p + 1) % NB
            wait_gather(slot)

            @pl.when(c + 1 < NCHUNKS)
            def _():
                wait_idx(nslot)
                issue_gather(nslot)

            @pl.when(c + 2 < NCHUNKS)
            def _():
                issue_idx(c + 2, slot)

            for r in range(CR):
                def tok(t, a, _slot=slot, _r=r):
                    ts = t * 8
                    a = list(a)
                    for k in range(8):
                        g = (k & 1) * VPR
                        for j in range(VPR):
                            a[g + j] = a[g + j] + rows_v[
                                _slot, _r, ts + k, pl.ds(j * 16, 16)]
                    return tuple(a)

                acc = lax.fori_loop(0, L // 8, tok, (zero,) * (2 * VPR))
                for j in range(VPR):
                    out_v[c * CR + r, pl.ds(j * 16, 16)] = (
                        acc[j] + acc[VPR + j])
        return _

    lax.fori_loop(0, NCHUNKS // NB, trip_body, None)
    pltpu.sync_copy(out_v, out_hbm.at[wid])


def _sc_pool(x, table):
    mesh = plsc.VectorSubcoreMesh(core_axis_name="c", subcore_axis_name="s")
    f = functools.partial(
        pl.kernel,
        out_type=jax.ShapeDtypeStruct((NW, ROWS_PER_W, EMBED), jnp.float32),
        mesh=mesh,
        scratch_types=[
            pltpu.VMEM((NB, CR, L), jnp.int32),
            pltpu.VMEM((NB, CR, L, EMBED), jnp.float32),
            pltpu.VMEM((ROWS_PER_W, EMBED), jnp.float32),
        ] + [pltpu.SemaphoreType.DMA] * (2 * NB),
        compiler_params=pltpu.CompilerParams(use_tc_tiling_on_sc=False),
    )(_sc_pool_body)
    return f(x, table)


def _mlp_body(e_ref, inv_ref, wh_ref, bh_ref, wf_ref, bf_ref, o_ref):
    e = e_ref[...] * inv_ref[...]
    e = jnp.where(e > 0, e, jnp.exp(e) - 1.0)
    h = lax.dot_general(e, wh_ref[...], (((1,), (1,)), ((), ())),
                        preferred_element_type=jnp.float32) + bh_ref[...]
    h = jnp.where(h > 0, h, jnp.exp(h) - 1.0)
    o = lax.dot_general(h, wf_ref[...], (((1,), (1,)), ((), ())),
                        preferred_element_type=jnp.float32) + bf_ref[...]
    m = jnp.max(o, axis=1, keepdims=True)
    o = o - m
    s = jnp.log(jnp.sum(jnp.exp(o), axis=1, keepdims=True))
    o_ref[...] = o - s


def _tc_mlp(pooled, inv_len, W_h, b_h, W_f, b_f):
    return pl.pallas_call(
        _mlp_body,
        out_shape=jax.ShapeDtypeStruct((B, NCLS), jnp.float32),
    )(pooled, inv_len, W_h, b_h, W_f, b_f)


def kernel(x, x_len, table, W_h, b_h, W_f, b_f):
    pooled = _sc_pool(x, table).reshape(B, EMBED)
    inv_len = (1.0 / x_len.astype(jnp.float32)).reshape(B, 1)
    return _tc_mlp(pooled, inv_len, W_h, b_h.reshape(1, HIDDEN),
                   W_f, b_f.reshape(1, NCLS))
